# baseline (speedup 1.0000x reference)
import functools

import jax
import jax.numpy as jnp
from jax import lax
from jax.experimental import pallas as pl
from jax.experimental.pallas import tpu as pltpu
from jax.experimental.pallas import tpu_sc as plsc

V_TILE = 2048


def _gather(table, idx):
    n, per_tok = idx.shape[0], table.shape[1]
    info = plsc.get_sparse_core_info()
    nw = info.num_cores * info.num_subcores
    b_per_w = n // nw
    mesh = plsc.VectorSubcoreMesh(core_axis_name="c", subcore_axis_name="s")

    chunk = 16

    @functools.partial(
        pl.kernel,
        out_type=jax.ShapeDtypeStruct((n, per_tok), jnp.float32),
        mesh=mesh,
        scratch_types=[
            pltpu.VMEM((b_per_w,), jnp.int32),
            pltpu.VMEM((b_per_w, per_tok), jnp.float32),
            pltpu.SemaphoreType.DMA,
        ],
    )
    def gather_kernel(idx_hbm, table_hbm, out_hbm, idx_s, rows_v, sem):
        wid = lax.axis_index("s") * info.num_cores + lax.axis_index("c")
        base = wid * b_per_w
        pltpu.sync_copy(idx_hbm.at[pl.ds(base, b_per_w)], idx_s)

        def body(ci, _):
            base_i = ci * chunk
            v = idx_s[pl.ds(base_i, chunk)]
            copies = []
            for j in range(chunk):
                copies.append(
                    pltpu.async_copy(
                        table_hbm.at[v[j]], rows_v.at[base_i + j], sem))
            for c in copies:
                c.wait()
            return ()

        lax.fori_loop(0, b_per_w // chunk, body, (), unroll=False)
        pltpu.sync_copy(rows_v, out_hbm.at[pl.ds(base, b_per_w)])

    return gather_kernel(idx, table)


def _h1_body(emb_ref, w1_ref, b1_ref, h1_ref):
    h1 = lax.dot_general(
        emb_ref[...], w1_ref[...], (((1,), (1,)), ((), ())),
        preferred_element_type=jnp.float32)
    h1_ref[...] = jnp.maximum(h1 + b1_ref[...], 0.0).astype(jnp.bfloat16)


def _h1(emb, W1, b1):
    batch = emb.shape[0]
    hid = W1.shape[0]
    return pl.pallas_call(
        _h1_body,
        out_shape=jax.ShapeDtypeStruct((batch, hid), jnp.bfloat16),
    )(emb, W1, b1.reshape(1, hid))


def _mm2_body(h1_ref, w2_ref, b2_ref, out_ref):
    w2b = w2_ref[...].astype(jnp.bfloat16)
    out_ref[...] = lax.dot_general(
        h1_ref[...], w2b, (((1,), (1,)), ((), ())),
        preferred_element_type=jnp.float32) + b2_ref[...]


def _mm2(h1b, W2, b2):
    batch, hid = h1b.shape
    vocab = W2.shape[0]
    return pl.pallas_call(
        _mm2_body,
        grid=(pl.cdiv(vocab, V_TILE),),
        in_specs=[
            pl.BlockSpec((batch, hid), lambda i: (0, 0)),
            pl.BlockSpec((V_TILE, hid), lambda i: (i, 0)),
            pl.BlockSpec((1, V_TILE), lambda i: (0, i)),
        ],
        out_specs=pl.BlockSpec((batch, V_TILE), lambda i: (0, i)),
        out_shape=jax.ShapeDtypeStruct((batch, vocab), jnp.float32),
    )(h1b, W2, b2.reshape(1, vocab))


def kernel(inputs, table, W1, b1, W2, b2):
    batch, ctx = inputs.shape
    idx = inputs.reshape(-1).astype(jnp.int32)
    emb = _gather(table, idx).reshape(batch, ctx * table.shape[1])
    h1b = _h1(emb, W1, b1)
    return _mm2(h1b, W2, b2)


# speedup vs baseline: 1.0091x; 1.0091x over previous
import functools

import jax
import jax.numpy as jnp
from jax import lax
from jax.experimental import pallas as pl
from jax.experimental.pallas import tpu as pltpu
from jax.experimental.pallas import tpu_sc as plsc

V_TILE = 4096


def _gather(table, idx):
    n, per_tok = idx.shape[0], table.shape[1]
    info = plsc.get_sparse_core_info()
    nw = info.num_cores * info.num_subcores
    b_per_w = n // nw
    mesh = plsc.VectorSubcoreMesh(core_axis_name="c", subcore_axis_name="s")

    chunk = 16

    @functools.partial(
        pl.kernel,
        out_type=jax.ShapeDtypeStruct((n, per_tok), jnp.float32),
        mesh=mesh,
        scratch_types=[
            pltpu.VMEM((b_per_w,), jnp.int32),
            pltpu.VMEM((b_per_w, per_tok), jnp.float32),
            pltpu.SemaphoreType.DMA,
        ],
    )
    def gather_kernel(idx_hbm, table_hbm, out_hbm, idx_s, rows_v, sem):
        wid = lax.axis_index("s") * info.num_cores + lax.axis_index("c")
        base = wid * b_per_w
        pltpu.sync_copy(idx_hbm.at[pl.ds(base, b_per_w)], idx_s)

        def body(ci, _):
            base_i = ci * chunk
            v = idx_s[pl.ds(base_i, chunk)]
            copies = []
            for j in range(chunk):
                copies.append(
                    pltpu.async_copy(
                        table_hbm.at[v[j]], rows_v.at[base_i + j], sem))
            for c in copies:
                c.wait()
            return ()

        lax.fori_loop(0, b_per_w // chunk, body, (), unroll=False)
        pltpu.sync_copy(rows_v, out_hbm.at[pl.ds(base, b_per_w)])

    return gather_kernel(idx, table)


def _h1_body(emb_ref, w1_ref, b1_ref, h1_ref):
    h1 = lax.dot_general(
        emb_ref[...], w1_ref[...], (((1,), (1,)), ((), ())),
        preferred_element_type=jnp.float32)
    h1_ref[...] = jnp.maximum(h1 + b1_ref[...], 0.0).astype(jnp.bfloat16)


def _h1(emb, W1, b1):
    batch = emb.shape[0]
    hid = W1.shape[0]
    return pl.pallas_call(
        _h1_body,
        out_shape=jax.ShapeDtypeStruct((batch, hid), jnp.bfloat16),
    )(emb, W1, b1.reshape(1, hid))


def _mm2_body(h1_ref, w2_ref, b2_ref, out_ref):
    w2b = w2_ref[...].astype(jnp.bfloat16)
    out_ref[...] = lax.dot_general(
        h1_ref[...], w2b, (((1,), (1,)), ((), ())),
        preferred_element_type=jnp.float32) + b2_ref[...]


def _mm2(h1b, W2, b2):
    batch, hid = h1b.shape
    vocab = W2.shape[0]
    return pl.pallas_call(
        _mm2_body,
        grid=(pl.cdiv(vocab, V_TILE),),
        in_specs=[
            pl.BlockSpec((batch, hid), lambda i: (0, 0)),
            pl.BlockSpec((V_TILE, hid), lambda i: (i, 0)),
            pl.BlockSpec((1, V_TILE), lambda i: (0, i)),
        ],
        out_specs=pl.BlockSpec((batch, V_TILE), lambda i: (0, i)),
        out_shape=jax.ShapeDtypeStruct((batch, vocab), jnp.float32),
    )(h1b, W2, b2.reshape(1, vocab))


def kernel(inputs, table, W1, b1, W2, b2):
    batch, ctx = inputs.shape
    idx = inputs.reshape(-1).astype(jnp.int32)
    emb = _gather(table, idx).reshape(batch, ctx * table.shape[1])
    h1b = _h1(emb, W1, b1)
    return _mm2(h1b, W2, b2)
